# baseline (speedup 1.0000x reference)
import functools

import jax
import jax.numpy as jnp
from jax import lax
from jax.experimental import pallas as pl
from jax.experimental.pallas import tpu as pltpu
from jax.experimental.pallas import tpu_sc as plsc

_BLOCK = 2048
_E = 8
_NT = (((1,), (1,)), ((), ()))
_NW = 32


def _score_kernel(rgb_ref, ir_ref, w_rgb_ref, b_rgb_ref, w_ir_ref, b_ir_ref,
                  sr_ref, si_ref):
    sr = lax.dot_general(rgb_ref[...], w_rgb_ref[...], _NT,
                         preferred_element_type=jnp.float32)
    si = lax.dot_general(ir_ref[...], w_ir_ref[...], _NT,
                         preferred_element_type=jnp.float32)
    sr_ref[...] = sr.T + b_rgb_ref[...].T
    si_ref[...] = si.T + b_ir_ref[...].T


def _sc_top2(v_ref, g):
    vs = [v_ref[e, pl.ds(g, 16)] for e in range(_E)]
    m1 = vs[0]
    i1 = jnp.zeros((16,), jnp.int32)
    for e in range(1, _E):
        gt = vs[e] > m1
        m1 = jnp.where(gt, vs[e], m1)
        i1 = jnp.where(gt, e, i1)
    m2 = jnp.full((16,), -jnp.inf, jnp.float32)
    i2 = jnp.full((16,), 127, jnp.int32)
    z = jnp.zeros((16,), jnp.float32)
    for e in range(_E):
        z = z + jnp.exp(vs[e] - m1)
        ve = jnp.where(i1 == e, -jnp.inf, vs[e])
        gt2 = ve > m2
        m2 = jnp.where(gt2, ve, m2)
        i2 = jnp.where(gt2, e, i2)
    rz = 1.0 / z
    return rz, jnp.exp(m2 - m1) * rz, i1, i2


def _make_sc_route(n):
    chunk = n // _NW
    mesh = plsc.VectorSubcoreMesh(core_axis_name="c", subcore_axis_name="s")

    @functools.partial(
        pl.kernel, mesh=mesh,
        out_type=[
            jax.ShapeDtypeStruct((4, n), jnp.float32),
            jax.ShapeDtypeStruct((2, n), jnp.int32),
            jax.ShapeDtypeStruct((2, n), jnp.int32),
        ],
        scratch_types=[
            pltpu.VMEM((_E, chunk), jnp.float32),
            pltpu.VMEM((_E, chunk), jnp.float32),
            pltpu.VMEM((4, chunk), jnp.float32),
            pltpu.VMEM((2, chunk), jnp.int32),
            pltpu.VMEM((2, chunk), jnp.int32),
            pltpu.SemaphoreType.DMA,
        ],
    )
    def sc_route(sr_hbm, si_hbm, probs_hbm, ixr_hbm, ixi_hbm,
                 sr_v, si_v, p_v, xr_v, xi_v, sem):
        wid = lax.axis_index("s") * 2 + lax.axis_index("c")
        base = wid * chunk
        pltpu.async_copy(sr_hbm.at[:, pl.ds(base, chunk)], sr_v, sem).wait()
        pltpu.async_copy(si_hbm.at[:, pl.ds(base, chunk)], si_v, sem).wait()

        @pl.loop(0, chunk, step=16)
        def _(g):
            p1r, p2r, i1r, i2r = _sc_top2(sr_v, g)
            p1i, p2i, i1i, i2i = _sc_top2(si_v, g)
            e1r, e2r = jnp.exp(p1r), jnp.exp(p2r)
            e1i, e2i = jnp.exp(p1i), jnp.exp(p2i)
            rden = 1.0 / (e1r + e2r + e1i + e2i)
            sl = pl.ds(g, 16)
            p_v[0, sl] = e1r * rden
            p_v[1, sl] = e2r * rden
            p_v[2, sl] = e1i * rden
            p_v[3, sl] = e2i * rden
            xr_v[0, sl] = i1r
            xr_v[1, sl] = i2r
            xi_v[0, sl] = i1i
            xi_v[1, sl] = i2i

        pltpu.async_copy(p_v, probs_hbm.at[:, pl.ds(base, chunk)], sem).wait()
        pltpu.async_copy(xr_v, ixr_hbm.at[:, pl.ds(base, chunk)], sem).wait()
        pltpu.async_copy(xi_v, ixi_hbm.at[:, pl.ds(base, chunk)], sem).wait()

    return sc_route


@functools.partial(jax.jit, static_argnames=("interpret",))
def kernel(rgb_local, ir_local, W_rgb, b_rgb, W_ir, b_ir, interpret=False):
    n = rgb_local.shape[0]
    d = rgb_local.shape[1]
    grid = n // _BLOCK

    row_spec = pl.BlockSpec((_BLOCK, d), lambda i: (i, 0))
    w_spec = pl.BlockSpec((_E, d), lambda i: (0, 0))
    b_spec = pl.BlockSpec((1, _E), lambda i: (0, 0))

    sr_t, si_t = pl.pallas_call(
        _score_kernel,
        grid=(grid,),
        in_specs=[row_spec, row_spec, w_spec, b_spec, w_spec, b_spec],
        out_specs=[
            pl.BlockSpec((_E, _BLOCK), lambda i: (0, i)),
            pl.BlockSpec((_E, _BLOCK), lambda i: (0, i)),
        ],
        out_shape=[
            jax.ShapeDtypeStruct((_E, n), jnp.float32),
            jax.ShapeDtypeStruct((_E, n), jnp.float32),
        ],
        interpret=interpret,
    )(rgb_local, ir_local, W_rgb, b_rgb.reshape(1, _E),
      W_ir, b_ir.reshape(1, _E))

    probs_t, idx_rgb_t, idx_ir_t = _make_sc_route(n)(sr_t, si_t)
    return probs_t.T, idx_rgb_t.T, idx_ir_t.T


# speedup vs baseline: 1.3934x; 1.3934x over previous
import functools

import jax
import jax.numpy as jnp
from jax.experimental import pallas as pl

_BLOCK = 2048
_E = 8
_BIG_I = 127
_NT = (((1,), (1,)), ((), ()))


def _route(x, w, bt):
    s = jax.lax.dot_general(x, w, _NT, preferred_element_type=jnp.float32)
    st = s.T + bt
    iota = jax.lax.broadcasted_iota(jnp.int32, st.shape, 0)
    m1 = jnp.max(st, axis=0, keepdims=True)
    i1 = jnp.min(jnp.where(st == m1, iota, _BIG_I), axis=0, keepdims=True)
    masked = jnp.where(iota == i1, -jnp.inf, st)
    m2 = jnp.max(masked, axis=0, keepdims=True)
    i2 = jnp.min(jnp.where(masked == m2, iota, _BIG_I), axis=0, keepdims=True)
    rz = 1.0 / jnp.sum(jnp.exp(st - m1), axis=0, keepdims=True)
    return rz, jnp.exp(m2 - m1) * rz, i1, i2


def _gate_kernel(rgb_ref, ir_ref, w_rgb_ref, b_rgb_ref, w_ir_ref, b_ir_ref,
                 probs_ref, idx_rgb_ref, idx_ir_ref):
    p1r, p2r, i1r, i2r = _route(rgb_ref[...], w_rgb_ref[...], b_rgb_ref[...].T)
    p1i, p2i, i1i, i2i = _route(ir_ref[...], w_ir_ref[...], b_ir_ref[...].T)

    e1r, e2r = jnp.exp(p1r), jnp.exp(p2r)
    e1i, e2i = jnp.exp(p1i), jnp.exp(p2i)
    rden = 1.0 / (e1r + e2r + e1i + e2i)
    probs_ref[...] = jnp.concatenate([e1r, e2r, e1i, e2i], axis=0) * rden
    idx_rgb_ref[...] = jnp.concatenate([i1r, i2r], axis=0)
    idx_ir_ref[...] = jnp.concatenate([i1i, i2i], axis=0)


@functools.partial(jax.jit, static_argnames=("interpret",))
def kernel(rgb_local, ir_local, W_rgb, b_rgb, W_ir, b_ir, interpret=False):
    n = rgb_local.shape[0]
    d = rgb_local.shape[1]
    grid = n // _BLOCK

    row_spec = pl.BlockSpec((_BLOCK, d), lambda i: (i, 0))
    w_spec = pl.BlockSpec((_E, d), lambda i: (0, 0))
    b_spec = pl.BlockSpec((1, _E), lambda i: (0, 0))

    probs_t, idx_rgb_t, idx_ir_t = pl.pallas_call(
        _gate_kernel,
        grid=(grid,),
        in_specs=[row_spec, row_spec, w_spec, b_spec, w_spec, b_spec],
        out_specs=[
            pl.BlockSpec((4, _BLOCK), lambda i: (0, i)),
            pl.BlockSpec((2, _BLOCK), lambda i: (0, i)),
            pl.BlockSpec((2, _BLOCK), lambda i: (0, i)),
        ],
        out_shape=[
            jax.ShapeDtypeStruct((4, n), jnp.float32),
            jax.ShapeDtypeStruct((2, n), jnp.int32),
            jax.ShapeDtypeStruct((2, n), jnp.int32),
        ],
        interpret=interpret,
    )(rgb_local, ir_local, W_rgb, b_rgb.reshape(1, _E),
      W_ir, b_ir.reshape(1, _E))
    return probs_t.T, idx_rgb_t.T, idx_ir_t.T
